# initial kernel scaffold (unmeasured)
import jax
import jax.numpy as jnp
from jax import lax
from jax.experimental import pallas as pl
from jax.experimental.pallas import tpu as pltpu

N_DEV = 8
M = 2048
N = 2048
CHUNK = M // N_DEV

_RANK_TABLE = (0, 1, 3, 2, 4, 5, 7, 6)


def kernel(A, B):
    m, k_per = A.shape
    _, n = B.shape
    assert (m, n) == (M, N)

    def body(a_ref, b_ref, out_ref, r0, r1, r2, rs_send, rs_recv, ag_send, ag_recv):
        my_pos = lax.axis_index("i")
        table = jnp.array(_RANK_TABLE, dtype=jnp.int32)
        r = table[my_pos]

        out_ref[:, :] = jnp.dot(
            a_ref[:, :].astype(jnp.bfloat16),
            b_ref[:, :].astype(jnp.bfloat16),
            preferred_element_type=jnp.float32,
        )

        recv_bufs = [r0, r1, r2]
        cur_start = jnp.int32(0)
        for s in range(3):
            keep = M >> (s + 1)
            bit = (r >> s) & 1
            partner = table[r ^ (1 << s)]
            send_start = cur_start + (1 - bit) * keep
            new_start = cur_start + bit * keep
            rdma = pltpu.make_async_remote_copy(
                src_ref=out_ref.at[pl.ds(send_start, keep), :],
                dst_ref=recv_bufs[s],
                send_sem=rs_send.at[s],
                recv_sem=rs_recv.at[s],
                device_id=(partner,),
                device_id_type=pl.DeviceIdType.MESH,
            )
            rdma.start()
            rdma.wait()
            out_ref[pl.ds(new_start, keep), :] = (
                out_ref[pl.ds(new_start, keep), :] + recv_bufs[s][:, :]
            )
            cur_start = new_start

        z = out_ref[pl.ds(cur_start, CHUNK), :]
        g = 0.5 * z * (1.0 + jnp.tanh(0.7978845608 * (z + 0.044715 * z * z * z)))
        out_ref[pl.ds(cur_start, CHUNK), :] = g

        own_start = cur_start
        for s in (2, 1, 0):
            size = M >> (s + 1)
            bit = (r >> s) & 1
            partner = table[r ^ (1 << s)]
            partner_start = own_start + (1 - 2 * bit) * size
            rdma = pltpu.make_async_remote_copy(
                src_ref=out_ref.at[pl.ds(own_start, size), :],
                dst_ref=out_ref.at[pl.ds(own_start, size), :],
                send_sem=ag_send.at[s],
                recv_sem=ag_recv.at[s],
                device_id=(partner,),
                device_id_type=pl.DeviceIdType.MESH,
            )
            rdma.start()
            rdma.wait()
            own_start = jnp.minimum(own_start, partner_start)

    return pl.pallas_call(
        body,
        out_shape=jax.ShapeDtypeStruct((M, N), jnp.float32),
        in_specs=[
            pl.BlockSpec(memory_space=pltpu.VMEM),
            pl.BlockSpec(memory_space=pltpu.VMEM),
        ],
        out_specs=pl.BlockSpec(memory_space=pltpu.VMEM),
        scratch_shapes=[
            pltpu.VMEM((M // 2, N), jnp.float32),
            pltpu.VMEM((M // 4, N), jnp.float32),
            pltpu.VMEM((M // 8, N), jnp.float32),
            pltpu.SemaphoreType.DMA((3,)),
            pltpu.SemaphoreType.DMA((3,)),
            pltpu.SemaphoreType.DMA((3,)),
            pltpu.SemaphoreType.DMA((3,)),
        ],
        compiler_params=pltpu.CompilerParams(collective_id=0),
    )(A, B)


# baseline (device time: 366950 ns/iter reference)
import jax
import jax.numpy as jnp
from jax import lax
from jax.experimental import pallas as pl
from jax.experimental.pallas import tpu as pltpu

N_DEV = 8
M = 2048
N = 2048
CHUNK = M // N_DEV

def _gray(p):
    return p ^ ((p >> 1) & 1)


def kernel(A, B):
    m, k_per = A.shape
    _, n = B.shape
    assert (m, n) == (M, N)

    def body(a_ref, b_ref, out_ref, r0, r1, r2, rs_send, rs_recv, ag_send, ag_recv):
        my_pos = lax.axis_index("i")
        r = _gray(my_pos)

        b_bf16 = b_ref[:, :].astype(jnp.bfloat16)
        for blk in range(0, M, 512):
            out_ref[blk : blk + 512, :] = jnp.dot(
                a_ref[blk : blk + 512, :].astype(jnp.bfloat16),
                b_bf16,
                preferred_element_type=jnp.float32,
            )

        recv_bufs = [r0, r1, r2]
        cur_start = jnp.int32(0)
        for s in range(3):
            keep = M >> (s + 1)
            bit = (r >> s) & 1
            partner = _gray(r ^ (1 << s))
            send_start = cur_start + (1 - bit) * keep
            new_start = cur_start + bit * keep
            rdma = pltpu.make_async_remote_copy(
                src_ref=out_ref.at[pl.ds(send_start, keep), :],
                dst_ref=recv_bufs[s],
                send_sem=rs_send.at[s],
                recv_sem=rs_recv.at[s],
                device_id=(partner,),
                device_id_type=pl.DeviceIdType.MESH,
            )
            rdma.start()
            rdma.wait()
            out_ref[pl.ds(new_start, keep), :] = (
                out_ref[pl.ds(new_start, keep), :] + recv_bufs[s][:, :]
            )
            cur_start = new_start

        z = out_ref[pl.ds(cur_start, CHUNK), :]
        g = 0.5 * z * (1.0 + jnp.tanh(0.7978845608 * (z + 0.044715 * z * z * z)))
        out_ref[pl.ds(cur_start, CHUNK), :] = g

        own_start = cur_start
        for s in (2, 1, 0):
            size = M >> (s + 1)
            bit = (r >> s) & 1
            partner = _gray(r ^ (1 << s))
            partner_start = own_start + (1 - 2 * bit) * size
            rdma = pltpu.make_async_remote_copy(
                src_ref=out_ref.at[pl.ds(own_start, size), :],
                dst_ref=out_ref.at[pl.ds(own_start, size), :],
                send_sem=ag_send.at[s],
                recv_sem=ag_recv.at[s],
                device_id=(partner,),
                device_id_type=pl.DeviceIdType.MESH,
            )
            rdma.start()
            rdma.wait()
            own_start = jnp.minimum(own_start, partner_start)

    return pl.pallas_call(
        body,
        out_shape=jax.ShapeDtypeStruct((M, N), jnp.float32),
        in_specs=[
            pl.BlockSpec(memory_space=pltpu.VMEM),
            pl.BlockSpec(memory_space=pltpu.VMEM),
        ],
        out_specs=pl.BlockSpec(memory_space=pltpu.VMEM),
        scratch_shapes=[
            pltpu.VMEM((M // 2, N), jnp.float32),
            pltpu.VMEM((M // 4, N), jnp.float32),
            pltpu.VMEM((M // 8, N), jnp.float32),
            pltpu.SemaphoreType.DMA((3,)),
            pltpu.SemaphoreType.DMA((3,)),
            pltpu.SemaphoreType.DMA((3,)),
            pltpu.SemaphoreType.DMA((3,)),
        ],
        compiler_params=pltpu.CompilerParams(
            vmem_limit_bytes=100 * 1024 * 1024,
        ),
    )(A, B)


# device time: 114132 ns/iter; 3.2151x vs baseline; 3.2151x over previous
import jax
import jax.numpy as jnp
from jax import lax
from jax.experimental import pallas as pl
from jax.experimental.pallas import tpu as pltpu

N_DEV = 8
M = 2048
N = 2048
CHUNK = M // N_DEV

COL_BOUNDS = (0, 768, 1408, 2048)
NG = 3
RS_RECV_OFF = (0, 1024, 1536)


def _gray(p):
    return p ^ ((p >> 1) & 1)


def kernel(A, B):
    m, k_per = A.shape
    _, n = B.shape
    assert (m, n) == (M, N)

    def body(
        a_ref,
        b_ref,
        out_ref,
        shadow,
        send0,
        send1,
        send2,
        recv0,
        recv1,
        recv2,
        rs_send,
        rs_recv,
        ag_send,
        ag_recv,
    ):
        my_pos = lax.axis_index("i")
        r = _gray(my_pos)
        sends = [send0, send1, send2]
        recvs = [recv0, recv1, recv2]

        b_bf16 = b_ref[:, :].astype(jnp.bfloat16)
        for blk in range(0, M, 512):
            out_ref[blk : blk + 512, :] = jnp.dot(
                a_ref[blk : blk + 512, :].astype(jnp.bfloat16),
                b_bf16,
                preferred_element_type=jnp.float32,
            )

        starts = [jnp.int32(0) for _ in range(NG)]
        for t in range(3):
            keep = M >> (t + 1)
            off = RS_RECV_OFF[t]
            rdmas = []
            for g in range(NG):
                s = (g + t) % 3
                c0, c1 = COL_BOUNDS[g], COL_BOUNDS[g + 1]
                bit = (r >> s) & 1
                partner = _gray(r ^ (1 << s))
                send_start = starts[g] + (1 - bit) * keep
                sends[g][0:keep, :] = out_ref[
                    pl.ds(send_start, keep), c0:c1
                ].astype(jnp.bfloat16)
                rdma = pltpu.make_async_remote_copy(
                    src_ref=sends[g].at[0:keep, :],
                    dst_ref=recvs[g].at[pl.ds(off, keep), :],
                    send_sem=rs_send.at[t, g],
                    recv_sem=rs_recv.at[t, g],
                    device_id=(partner,),
                    device_id_type=pl.DeviceIdType.MESH,
                )
                rdma.start()
                rdmas.append(rdma)
                starts[g] = starts[g] + bit * keep
            for g in range(NG):
                rdmas[g].wait()
                c0, c1 = COL_BOUNDS[g], COL_BOUNDS[g + 1]
                out_ref[pl.ds(starts[g], keep), c0:c1] = (
                    out_ref[pl.ds(starts[g], keep), c0:c1]
                    + recvs[g][pl.ds(off, keep), :].astype(jnp.float32)
                )

        for g in range(NG):
            c0, c1 = COL_BOUNDS[g], COL_BOUNDS[g + 1]
            z = out_ref[pl.ds(starts[g], CHUNK), c0:c1]
            gz = 0.5 * z * (1.0 + jnp.tanh(0.7978845608 * (z + 0.044715 * z * z * z)))
            shadow[pl.ds(starts[g], CHUNK), c0:c1] = gz.astype(jnp.bfloat16)

        own = list(starts)
        for t in range(3):
            size = CHUNK << t
            rdmas = []
            partner_starts = []
            for g in range(NG):
                s = (g + 2 - t) % 3
                c0, c1 = COL_BOUNDS[g], COL_BOUNDS[g + 1]
                bit = (r >> s) & 1
                partner = _gray(r ^ (1 << s))
                partner_starts.append(own[g] + (1 - 2 * bit) * size)
                rdma = pltpu.make_async_remote_copy(
                    src_ref=shadow.at[pl.ds(own[g], size), c0:c1],
                    dst_ref=shadow.at[pl.ds(own[g], size), c0:c1],
                    send_sem=ag_send.at[t, g],
                    recv_sem=ag_recv.at[t, g],
                    device_id=(partner,),
                    device_id_type=pl.DeviceIdType.MESH,
                )
                rdma.start()
                rdmas.append(rdma)
            for g in range(NG):
                rdmas[g].wait()
                own[g] = jnp.minimum(own[g], partner_starts[g])

        for blk in range(0, M, 512):
            out_ref[blk : blk + 512, :] = shadow[blk : blk + 512, :].astype(
                jnp.float32
            )

    wg = [COL_BOUNDS[g + 1] - COL_BOUNDS[g] for g in range(NG)]
    return pl.pallas_call(
        body,
        out_shape=jax.ShapeDtypeStruct((M, N), jnp.float32),
        in_specs=[
            pl.BlockSpec(memory_space=pltpu.VMEM),
            pl.BlockSpec(memory_space=pltpu.VMEM),
        ],
        out_specs=pl.BlockSpec(memory_space=pltpu.VMEM),
        scratch_shapes=[
            pltpu.VMEM((M, N), jnp.bfloat16),
            pltpu.VMEM((M // 2, wg[0]), jnp.bfloat16),
            pltpu.VMEM((M // 2, wg[1]), jnp.bfloat16),
            pltpu.VMEM((M // 2, wg[2]), jnp.bfloat16),
            pltpu.VMEM((1792, wg[0]), jnp.bfloat16),
            pltpu.VMEM((1792, wg[1]), jnp.bfloat16),
            pltpu.VMEM((1792, wg[2]), jnp.bfloat16),
            pltpu.SemaphoreType.DMA((3, 3)),
            pltpu.SemaphoreType.DMA((3, 3)),
            pltpu.SemaphoreType.DMA((3, 3)),
            pltpu.SemaphoreType.DMA((3, 3)),
        ],
        compiler_params=pltpu.CompilerParams(
            vmem_limit_bytes=100 * 1024 * 1024,
        ),
    )(A, B)


# device time: 99417 ns/iter; 3.6910x vs baseline; 1.1480x over previous
import jax
import jax.numpy as jnp
from jax import lax
from jax.experimental import pallas as pl
from jax.experimental.pallas import tpu as pltpu

N_DEV = 8
M = 2048
N = 2048
CHUNK = M // N_DEV

COL_BOUNDS = (0, 768, 1408, 2048)
NG = 3
RS_RECV_OFF = (0, 1024, 1536)


def _gray(p):
    return p ^ ((p >> 1) & 1)


def kernel(A, B):
    m, k_per = A.shape
    _, n = B.shape
    assert (m, n) == (M, N)

    def body(
        a_ref,
        b_ref,
        out_ref,
        acc,
        send0,
        send1,
        send2,
        recv0,
        recv1,
        recv2,
        rs_send,
        rs_recv,
        ag_send,
        ag_recv,
    ):
        my_pos = lax.axis_index("i")
        r = _gray(my_pos)
        sends = [send0, send1, send2]
        recvs = [recv0, recv1, recv2]

        b_bf16 = b_ref[:, :].astype(jnp.bfloat16)

        starts = [jnp.int32(0) for _ in range(NG)]
        keep0 = M // 2
        rdmas = []
        for g in range(NG):
            c0, c1 = COL_BOUNDS[g], COL_BOUNDS[g + 1]
            bit = (r >> g) & 1
            partner = _gray(r ^ (1 << g))
            send_start = (1 - bit) * keep0
            acc[pl.ds(send_start, keep0), c0:c1] = jnp.dot(
                a_ref[pl.ds(send_start, keep0), :].astype(jnp.bfloat16),
                b_bf16[:, c0:c1],
                preferred_element_type=jnp.float32,
            )
            sends[g][0:keep0, :] = acc[pl.ds(send_start, keep0), c0:c1].astype(
                jnp.bfloat16
            )
            rdma = pltpu.make_async_remote_copy(
                src_ref=sends[g].at[0:keep0, :],
                dst_ref=recvs[g].at[pl.ds(RS_RECV_OFF[0], keep0), :],
                send_sem=rs_send.at[0, g],
                recv_sem=rs_recv.at[0, g],
                device_id=(partner,),
                device_id_type=pl.DeviceIdType.MESH,
            )
            rdma.start()
            rdmas.append(rdma)
            starts[g] = starts[g] + bit * keep0
        for g in range(NG):
            c0, c1 = COL_BOUNDS[g], COL_BOUNDS[g + 1]
            acc[pl.ds(starts[g], keep0), c0:c1] = jnp.dot(
                a_ref[pl.ds(starts[g], keep0), :].astype(jnp.bfloat16),
                b_bf16[:, c0:c1],
                preferred_element_type=jnp.float32,
            )
        for g in range(NG):
            rdmas[g].wait()
            c0, c1 = COL_BOUNDS[g], COL_BOUNDS[g + 1]
            acc[pl.ds(starts[g], keep0), c0:c1] = (
                acc[pl.ds(starts[g], keep0), c0:c1]
                + recvs[g][pl.ds(RS_RECV_OFF[0], keep0), :].astype(jnp.float32)
            )

        for t in range(1, 3):
            keep = M >> (t + 1)
            off = RS_RECV_OFF[t]
            rdmas = []
            for g in range(NG):
                s = (g + t) % 3
                c0, c1 = COL_BOUNDS[g], COL_BOUNDS[g + 1]
                bit = (r >> s) & 1
                partner = _gray(r ^ (1 << s))
                send_start = starts[g] + (1 - bit) * keep
                sends[g][0:keep, :] = acc[pl.ds(send_start, keep), c0:c1].astype(
                    jnp.bfloat16
                )
                rdma = pltpu.make_async_remote_copy(
                    src_ref=sends[g].at[0:keep, :],
                    dst_ref=recvs[g].at[pl.ds(off, keep), :],
                    send_sem=rs_send.at[t, g],
                    recv_sem=rs_recv.at[t, g],
                    device_id=(partner,),
                    device_id_type=pl.DeviceIdType.MESH,
                )
                rdma.start()
                rdmas.append(rdma)
                starts[g] = starts[g] + bit * keep
            for g in range(NG):
                rdmas[g].wait()
                c0, c1 = COL_BOUNDS[g], COL_BOUNDS[g + 1]
                acc[pl.ds(starts[g], keep), c0:c1] = (
                    acc[pl.ds(starts[g], keep), c0:c1]
                    + recvs[g][pl.ds(off, keep), :].astype(jnp.float32)
                )

        for g in range(NG):
            c0, c1 = COL_BOUNDS[g], COL_BOUNDS[g + 1]
            z = acc[pl.ds(starts[g], CHUNK), c0:c1]
            gz = 0.5 * z * (1.0 + jnp.tanh(0.7978845608 * (z + 0.044715 * z * z * z)))
            out_ref[pl.ds(starts[g], CHUNK), c0:c1] = gz.astype(jnp.bfloat16)

        own = list(starts)
        for t in range(3):
            size = CHUNK << t
            rdmas = []
            partner_starts = []
            for g in range(NG):
                s = (g + 2 - t) % 3
                c0, c1 = COL_BOUNDS[g], COL_BOUNDS[g + 1]
                bit = (r >> s) & 1
                partner = _gray(r ^ (1 << s))
                partner_starts.append(own[g] + (1 - 2 * bit) * size)
                rdma = pltpu.make_async_remote_copy(
                    src_ref=out_ref.at[pl.ds(own[g], size), c0:c1],
                    dst_ref=out_ref.at[pl.ds(own[g], size), c0:c1],
                    send_sem=ag_send.at[t, g],
                    recv_sem=ag_recv.at[t, g],
                    device_id=(partner,),
                    device_id_type=pl.DeviceIdType.MESH,
                )
                rdma.start()
                rdmas.append(rdma)
            for g in range(NG):
                rdmas[g].wait()
                own[g] = jnp.minimum(own[g], partner_starts[g])

    wg = [COL_BOUNDS[g + 1] - COL_BOUNDS[g] for g in range(NG)]
    return pl.pallas_call(
        body,
        out_shape=jax.ShapeDtypeStruct((M, N), jnp.bfloat16),
        in_specs=[
            pl.BlockSpec(memory_space=pltpu.VMEM),
            pl.BlockSpec(memory_space=pltpu.VMEM),
        ],
        out_specs=pl.BlockSpec(memory_space=pltpu.VMEM),
        scratch_shapes=[
            pltpu.VMEM((M, N), jnp.float32),
            pltpu.VMEM((M // 2, wg[0]), jnp.bfloat16),
            pltpu.VMEM((M // 2, wg[1]), jnp.bfloat16),
            pltpu.VMEM((M // 2, wg[2]), jnp.bfloat16),
            pltpu.VMEM((1792, wg[0]), jnp.bfloat16),
            pltpu.VMEM((1792, wg[1]), jnp.bfloat16),
            pltpu.VMEM((1792, wg[2]), jnp.bfloat16),
            pltpu.SemaphoreType.DMA((3, 3)),
            pltpu.SemaphoreType.DMA((3, 3)),
            pltpu.SemaphoreType.DMA((3, 3)),
            pltpu.SemaphoreType.DMA((3, 3)),
        ],
        compiler_params=pltpu.CompilerParams(
            vmem_limit_bytes=100 * 1024 * 1024,
        ),
    )(A, B)


# device time: 86845 ns/iter; 4.2253x vs baseline; 1.1448x over previous
import jax
import jax.numpy as jnp
from jax import lax
from jax.experimental import pallas as pl
from jax.experimental.pallas import tpu as pltpu

N_DEV = 8
M = 2048
N = 2048
CHUNK = M // N_DEV

COL_BOUNDS = (0, 768, 1408, 2048)
NG = 3
RS_RECV_OFF = (0, 1024, 1536)


def _gray(p):
    return p ^ ((p >> 1) & 1)


def kernel(A, B):
    m, k_per = A.shape
    _, n = B.shape
    assert (m, n) == (M, N)

    def body(
        a_ref,
        b_ref,
        out_ref,
        acc,
        send0,
        send1,
        send2,
        recv0,
        recv1,
        recv2,
        rs_send,
        rs_recv,
        ag_send,
        ag_recv,
    ):
        my_pos = lax.axis_index("i")
        r = _gray(my_pos)
        sends = [send0, send1, send2]
        recvs = [recv0, recv1, recv2]
        cols = [(COL_BOUNDS[g], COL_BOUNDS[g + 1]) for g in range(NG)]
        bits = [[(r >> ((g + t) % 3)) & 1 for t in range(3)] for g in range(NG)]
        nbrs = [
            [_gray(r ^ (1 << ((g + t) % 3))) for t in range(3)] for g in range(NG)
        ]

        barrier_sem = pltpu.get_barrier_semaphore()
        for s in range(3):
            pl.semaphore_signal(
                barrier_sem,
                inc=1,
                device_id=(_gray(r ^ (1 << s)),),
                device_id_type=pl.DeviceIdType.MESH,
            )
        pl.semaphore_wait(barrier_sem, 3)

        b_bf16 = b_ref[:, :].astype(jnp.bfloat16)

        starts = [bits[g][0] * (M // 2) for g in range(NG)]
        rdma0 = [[None, None] for _ in range(NG)]
        for piece in range(2):
            for g in range(NG):
                c0, c1 = cols[g]
                send_start = (1 - bits[g][0]) * (M // 2) + piece * 512
                acc[pl.ds(send_start, 512), c0:c1] = jnp.dot(
                    a_ref[pl.ds(send_start, 512), :].astype(jnp.bfloat16),
                    b_bf16[:, c0:c1],
                    preferred_element_type=jnp.float32,
                )
                sends[g][piece * 512 : (piece + 1) * 512, :] = acc[
                    pl.ds(send_start, 512), c0:c1
                ].astype(jnp.bfloat16)
                rd = pltpu.make_async_remote_copy(
                    src_ref=sends[g].at[piece * 512 : (piece + 1) * 512, :],
                    dst_ref=recvs[g].at[piece * 512 : (piece + 1) * 512, :],
                    send_sem=rs_send.at[piece, g],
                    recv_sem=rs_recv.at[piece, g],
                    device_id=(nbrs[g][0],),
                    device_id_type=pl.DeviceIdType.MESH,
                )
                rd.start()
                rdma0[g][piece] = rd
        for g in range(NG):
            c0, c1 = cols[g]
            acc[pl.ds(starts[g], M // 2), c0:c1] = jnp.dot(
                a_ref[pl.ds(starts[g], M // 2), :].astype(jnp.bfloat16),
                b_bf16[:, c0:c1],
                preferred_element_type=jnp.float32,
            )

        rdma1 = [None] * NG
        for g in range(NG):
            c0, c1 = cols[g]
            rdma0[g][0].wait()
            rdma0[g][1].wait()
            b1 = bits[g][1]
            fwd_rel = (1 - b1) * 512
            fwd_abs = starts[g] + fwd_rel
            sfwd = acc[pl.ds(fwd_abs, 512), c0:c1] + recvs[g][
                pl.ds(fwd_rel, 512), :
            ].astype(jnp.float32)
            acc[pl.ds(fwd_abs, 512), c0:c1] = sfwd
            sends[g][0:512, :] = sfwd.astype(jnp.bfloat16)
            rd = pltpu.make_async_remote_copy(
                src_ref=sends[g].at[0:512, :],
                dst_ref=recvs[g].at[pl.ds(RS_RECV_OFF[1], 512), :],
                send_sem=rs_send.at[2, g],
                recv_sem=rs_recv.at[2, g],
                device_id=(nbrs[g][1],),
                device_id_type=pl.DeviceIdType.MESH,
            )
            rd.start()
            rdma1[g] = rd
            keep_rel = b1 * 512
            keep_abs = starts[g] + keep_rel
            acc[pl.ds(keep_abs, 512), c0:c1] = acc[
                pl.ds(keep_abs, 512), c0:c1
            ] + recvs[g][pl.ds(keep_rel, 512), :].astype(jnp.float32)
            starts[g] = keep_abs

        rdma2 = [None] * NG
        for g in range(NG):
            c0, c1 = cols[g]
            rdma1[g].wait()
            b2 = bits[g][2]
            fwd_rel = (1 - b2) * 256
            fwd_abs = starts[g] + fwd_rel
            sfwd = acc[pl.ds(fwd_abs, 256), c0:c1] + recvs[g][
                pl.ds(RS_RECV_OFF[1] + fwd_rel, 256), :
            ].astype(jnp.float32)
            acc[pl.ds(fwd_abs, 256), c0:c1] = sfwd
            sends[g][0:256, :] = sfwd.astype(jnp.bfloat16)
            rd = pltpu.make_async_remote_copy(
                src_ref=sends[g].at[0:256, :],
                dst_ref=recvs[g].at[pl.ds(RS_RECV_OFF[2], 256), :],
                send_sem=rs_send.at[3, g],
                recv_sem=rs_recv.at[3, g],
                device_id=(nbrs[g][2],),
                device_id_type=pl.DeviceIdType.MESH,
            )
            rd.start()
            rdma2[g] = rd
            keep_rel = b2 * 256
            keep_abs = starts[g] + keep_rel
            acc[pl.ds(keep_abs, 256), c0:c1] = acc[
                pl.ds(keep_abs, 256), c0:c1
            ] + recvs[g][pl.ds(RS_RECV_OFF[1] + keep_rel, 256), :].astype(
                jnp.float32
            )
            starts[g] = keep_abs

        ag_rd = [[None] * 5 for _ in range(NG)]

        def _ag_piece(g, idx, row_start, nrows, target):
            c0, c1 = cols[g]
            rd = pltpu.make_async_remote_copy(
                src_ref=out_ref.at[pl.ds(row_start, nrows), c0:c1],
                dst_ref=out_ref.at[pl.ds(row_start, nrows), c0:c1],
                send_sem=ag_send.at[idx, g],
                recv_sem=ag_recv.at[idx, g],
                device_id=(target,),
                device_id_type=pl.DeviceIdType.MESH,
            )
            rd.start()
            ag_rd[g][idx] = rd

        for g in range(NG):
            c0, c1 = cols[g]
            rdma2[g].wait()
            z = acc[pl.ds(starts[g], CHUNK), c0:c1] + recvs[g][
                pl.ds(RS_RECV_OFF[2], CHUNK), :
            ].astype(jnp.float32)
            gz = 0.5 * z * (1.0 + jnp.tanh(0.7978845608 * (z + 0.044715 * z * z * z)))
            out_ref[pl.ds(starts[g], CHUNK), c0:c1] = gz.astype(jnp.bfloat16)
            _ag_piece(g, 0, starts[g], CHUNK, nbrs[g][2])
            _ag_piece(g, 1, starts[g], CHUNK, nbrs[g][1])

        for g in range(NG):
            chunk = starts[g]
            sib256 = chunk + (1 - 2 * bits[g][2]) * 256
            b512 = chunk - bits[g][2] * 256
            ag_rd[g][0].wait()
            _ag_piece(g, 2, sib256, CHUNK, nbrs[g][1])
            _ag_piece(g, 3, b512, 512, nbrs[g][0])

        for g in range(NG):
            b512 = starts[g] - bits[g][2] * 256
            sib512 = b512 + (1 - 2 * bits[g][1]) * 512
            ag_rd[g][1].wait()
            ag_rd[g][2].wait()
            _ag_piece(g, 4, sib512, 512, nbrs[g][0])

        for g in range(NG):
            ag_rd[g][3].wait()
            ag_rd[g][4].wait()

    wg = [COL_BOUNDS[g + 1] - COL_BOUNDS[g] for g in range(NG)]
    return pl.pallas_call(
        body,
        out_shape=jax.ShapeDtypeStruct((M, N), jnp.bfloat16),
        in_specs=[
            pl.BlockSpec(memory_space=pltpu.VMEM),
            pl.BlockSpec(memory_space=pltpu.VMEM),
        ],
        out_specs=pl.BlockSpec(memory_space=pltpu.VMEM),
        scratch_shapes=[
            pltpu.VMEM((M, N), jnp.float32),
            pltpu.VMEM((M // 2, wg[0]), jnp.bfloat16),
            pltpu.VMEM((M // 2, wg[1]), jnp.bfloat16),
            pltpu.VMEM((M // 2, wg[2]), jnp.bfloat16),
            pltpu.VMEM((1792, wg[0]), jnp.bfloat16),
            pltpu.VMEM((1792, wg[1]), jnp.bfloat16),
            pltpu.VMEM((1792, wg[2]), jnp.bfloat16),
            pltpu.SemaphoreType.DMA((4, 3)),
            pltpu.SemaphoreType.DMA((4, 3)),
            pltpu.SemaphoreType.DMA((5, 3)),
            pltpu.SemaphoreType.DMA((5, 3)),
        ],
        compiler_params=pltpu.CompilerParams(
            vmem_limit_bytes=100 * 1024 * 1024,
            collective_id=0,
        ),
    )(A, B)


# device time: 83339 ns/iter; 4.4031x vs baseline; 1.0421x over previous
import jax
import jax.numpy as jnp
from jax import lax
from jax.experimental import pallas as pl
from jax.experimental.pallas import tpu as pltpu

N_DEV = 8
M = 2048
N = 2048
CHUNK = M // N_DEV

COL_BOUNDS = (0, 768, 1408, 2048)
NG = 3
T1_OFF = 1024
T2_OFF = 1536


def _gray(p):
    return p ^ ((p >> 1) & 1)


def kernel(A, B):
    m, k_per = A.shape
    _, n = B.shape
    assert (m, n) == (M, N)

    def body(
        a_ref,
        b_ref,
        out_ref,
        acc,
        send0,
        send1,
        send2,
        recv0,
        recv1,
        recv2,
        rs_send,
        rs_recv,
        ag_send,
        ag_recv,
    ):
        my_pos = lax.axis_index("i")
        r = _gray(my_pos)
        sends = [send0, send1, send2]
        recvs = [recv0, recv1, recv2]
        cols = [(COL_BOUNDS[g], COL_BOUNDS[g + 1]) for g in range(NG)]
        bits = [[(r >> ((g + t) % 3)) & 1 for t in range(3)] for g in range(NG)]
        nbrs = [
            [_gray(r ^ (1 << ((g + t) % 3))) for t in range(3)] for g in range(NG)
        ]
        send_base = [(1 - bits[g][0]) * (M // 2) for g in range(NG)]
        starts0 = [bits[g][0] * (M // 2) for g in range(NG)]
        fwdrel = [(1 - bits[g][1]) * 512 for g in range(NG)]
        keeprel = [bits[g][1] * 512 for g in range(NG)]
        f2 = [(1 - bits[g][2]) * 256 for g in range(NG)]
        k2 = [bits[g][2] * 256 for g in range(NG)]
        starts1 = [starts0[g] + keeprel[g] for g in range(NG)]
        starts2 = [starts1[g] + k2[g] for g in range(NG)]
        fwd2rel = [keeprel[g] + f2[g] for g in range(NG)]
        keep2rel = [keeprel[g] + k2[g] for g in range(NG)]

        barrier_sem = pltpu.get_barrier_semaphore()
        for s in range(3):
            pl.semaphore_signal(
                barrier_sem,
                inc=1,
                device_id=(_gray(r ^ (1 << s)),),
                device_id_type=pl.DeviceIdType.MESH,
            )
        pl.semaphore_wait(barrier_sem, 3)

        b_bf16 = b_ref[:, :].astype(jnp.bfloat16)

        def _mm(row_start, nrows, g):
            c0, c1 = cols[g]
            return jnp.dot(
                a_ref[pl.ds(row_start, nrows), :].astype(jnp.bfloat16),
                b_bf16[:, c0:c1],
                preferred_element_type=jnp.float32,
            )

        def _rs(g, slot, stage_rows, src_rows, dst_row, nrows):
            rd = pltpu.make_async_remote_copy(
                src_ref=sends[g].at[src_rows[0] : src_rows[1], :],
                dst_ref=recvs[g].at[pl.ds(dst_row, nrows), :],
                send_sem=rs_send.at[slot, g],
                recv_sem=rs_recv.at[slot, g],
                device_id=(nbrs[g][stage_rows],),
                device_id_type=pl.DeviceIdType.MESH,
            )
            rd.start()
            return rd

        p0 = [None] * NG
        for g in range(NG):
            c0, c1 = cols[g]
            rows = send_base[g] + fwdrel[g]
            sends[g][0:512, :] = _mm(rows, 512, g).astype(jnp.bfloat16)
            p0[g] = _rs(g, 0, 0, (0, 512), fwdrel[g], 512)

        for g in range(NG):
            c0, c1 = cols[g]
            acc[pl.ds(starts0[g] + fwdrel[g], 512), c0:c1] = _mm(
                starts0[g] + fwdrel[g], 512, g
            )

        p3 = [None] * NG
        p4 = [None] * NG
        for g in range(NG):
            c0, c1 = cols[g]
            p0[g].wait()
            base = starts0[g] + fwdrel[g]
            pa = acc[pl.ds(base + f2[g], 256), c0:c1] + recvs[g][
                pl.ds(fwdrel[g] + f2[g], 256), :
            ].astype(jnp.float32)
            sends[g][0:256, :] = pa.astype(jnp.bfloat16)
            p3[g] = _rs(g, 3, 1, (0, 256), T1_OFF + f2[g], 256)
            pb = acc[pl.ds(base + k2[g], 256), c0:c1] + recvs[g][
                pl.ds(fwdrel[g] + k2[g], 256), :
            ].astype(jnp.float32)
            sends[g][256:512, :] = pb.astype(jnp.bfloat16)
            p4[g] = _rs(g, 4, 1, (256, 512), T1_OFF + k2[g], 256)

        p1 = [None] * NG
        p2 = [None] * NG
        for g in range(NG):
            sends[g][512:768, :] = _mm(
                send_base[g] + fwd2rel[g], 256, g
            ).astype(jnp.bfloat16)
            p1[g] = _rs(g, 1, 0, (512, 768), fwd2rel[g], 256)
            sends[g][768:1024, :] = _mm(
                send_base[g] + keep2rel[g], 256, g
            ).astype(jnp.bfloat16)
            p2[g] = _rs(g, 2, 0, (768, 1024), keep2rel[g], 256)

        for g in range(NG):
            c0, c1 = cols[g]
            acc[pl.ds(starts1[g], 512), c0:c1] = _mm(starts1[g], 512, g)

        for g in range(NG):
            c0, c1 = cols[g]
            p1[g].wait()
            rows = starts1[g] + f2[g]
            acc[pl.ds(rows, 256), c0:c1] = acc[pl.ds(rows, 256), c0:c1] + recvs[
                g
            ][pl.ds(fwd2rel[g], 256), :].astype(jnp.float32)

        p5 = [None] * NG
        for g in range(NG):
            c0, c1 = cols[g]
            p3[g].wait()
            rows = starts1[g] + f2[g]
            s2 = acc[pl.ds(rows, 256), c0:c1] + recvs[g][
                pl.ds(T1_OFF + f2[g], 256), :
            ].astype(jnp.float32)
            sends[g][0:256, :] = s2.astype(jnp.bfloat16)
            p5[g] = _rs(g, 5, 2, (0, 256), T2_OFF, 256)

        for g in range(NG):
            c0, c1 = cols[g]
            p2[g].wait()
            acc[pl.ds(starts2[g], 256), c0:c1] = acc[
                pl.ds(starts2[g], 256), c0:c1
            ] + recvs[g][pl.ds(keep2rel[g], 256), :].astype(jnp.float32)
        for g in range(NG):
            c0, c1 = cols[g]
            p4[g].wait()
            acc[pl.ds(starts2[g], 256), c0:c1] = acc[
                pl.ds(starts2[g], 256), c0:c1
            ] + recvs[g][pl.ds(T1_OFF + k2[g], 256), :].astype(jnp.float32)

        ag_rd = [[None] * 5 for _ in range(NG)]

        def _ag_piece(g, idx, row_start, nrows, target):
            c0, c1 = cols[g]
            rd = pltpu.make_async_remote_copy(
                src_ref=out_ref.at[pl.ds(row_start, nrows), c0:c1],
                dst_ref=out_ref.at[pl.ds(row_start, nrows), c0:c1],
                send_sem=ag_send.at[idx, g],
                recv_sem=ag_recv.at[idx, g],
                device_id=(target,),
                device_id_type=pl.DeviceIdType.MESH,
            )
            rd.start()
            ag_rd[g][idx] = rd

        for g in range(NG):
            c0, c1 = cols[g]
            p5[g].wait()
            z = acc[pl.ds(starts2[g], CHUNK), c0:c1] + recvs[g][
                pl.ds(T2_OFF, CHUNK), :
            ].astype(jnp.float32)
            gz = 0.5 * z * (1.0 + jnp.tanh(0.7978845608 * (z + 0.044715 * z * z * z)))
            out_ref[pl.ds(starts2[g], CHUNK), c0:c1] = gz.astype(jnp.bfloat16)
            _ag_piece(g, 0, starts2[g], CHUNK, nbrs[g][2])
            _ag_piece(g, 1, starts2[g], CHUNK, nbrs[g][1])

        for g in range(NG):
            chunk = starts2[g]
            sib256 = chunk + (1 - 2 * bits[g][2]) * 256
            b512 = chunk - bits[g][2] * 256
            ag_rd[g][0].wait()
            _ag_piece(g, 2, sib256, CHUNK, nbrs[g][1])
            _ag_piece(g, 3, b512, 512, nbrs[g][0])

        for g in range(NG):
            b512 = starts2[g] - bits[g][2] * 256
            sib512 = b512 + (1 - 2 * bits[g][1]) * 512
            ag_rd[g][1].wait()
            ag_rd[g][2].wait()
            _ag_piece(g, 4, sib512, 512, nbrs[g][0])

        for g in range(NG):
            ag_rd[g][3].wait()
            ag_rd[g][4].wait()

    wg = [COL_BOUNDS[g + 1] - COL_BOUNDS[g] for g in range(NG)]
    return pl.pallas_call(
        body,
        out_shape=jax.ShapeDtypeStruct((M, N), jnp.bfloat16),
        in_specs=[
            pl.BlockSpec(memory_space=pltpu.VMEM),
            pl.BlockSpec(memory_space=pltpu.VMEM),
        ],
        out_specs=pl.BlockSpec(memory_space=pltpu.VMEM),
        scratch_shapes=[
            pltpu.VMEM((M, N), jnp.float32),
            pltpu.VMEM((M // 2, wg[0]), jnp.bfloat16),
            pltpu.VMEM((M // 2, wg[1]), jnp.bfloat16),
            pltpu.VMEM((M // 2, wg[2]), jnp.bfloat16),
            pltpu.VMEM((1792, wg[0]), jnp.bfloat16),
            pltpu.VMEM((1792, wg[1]), jnp.bfloat16),
            pltpu.VMEM((1792, wg[2]), jnp.bfloat16),
            pltpu.SemaphoreType.DMA((6, 3)),
            pltpu.SemaphoreType.DMA((6, 3)),
            pltpu.SemaphoreType.DMA((5, 3)),
            pltpu.SemaphoreType.DMA((5, 3)),
        ],
        compiler_params=pltpu.CompilerParams(
            vmem_limit_bytes=100 * 1024 * 1024,
            collective_id=0,
        ),
    )(A, B)


# device time: 81989 ns/iter; 4.4756x vs baseline; 1.0165x over previous
import jax
import jax.numpy as jnp
from jax import lax
from jax.experimental import pallas as pl
from jax.experimental.pallas import tpu as pltpu

N_DEV = 8
M = 2048
N = 2048
CHUNK = M // N_DEV

COL_BOUNDS = (0, 768, 1408, 2048)
NG = 3
T1_OFF = 1024
T2_OFF = 1536


def _gray(p):
    return p ^ ((p >> 1) & 1)


def kernel(A, B):
    m, k_per = A.shape
    _, n = B.shape
    assert (m, n) == (M, N)

    def body(
        a_ref,
        b_ref,
        out_ref,
        acc,
        send0,
        send1,
        send2,
        recv0,
        recv1,
        recv2,
        rs_send,
        rs_recv,
        ag_send,
        ag_recv,
    ):
        my_pos = lax.axis_index("i")
        r = _gray(my_pos)
        sends = [send0, send1, send2]
        recvs = [recv0, recv1, recv2]
        cols = [(COL_BOUNDS[g], COL_BOUNDS[g + 1]) for g in range(NG)]
        bits = [[(r >> ((g + t) % 3)) & 1 for t in range(3)] for g in range(NG)]
        nbrs = [
            [_gray(r ^ (1 << ((g + t) % 3))) for t in range(3)] for g in range(NG)
        ]
        send_base = [(1 - bits[g][0]) * (M // 2) for g in range(NG)]
        starts0 = [bits[g][0] * (M // 2) for g in range(NG)]
        fwdrel = [(1 - bits[g][1]) * 512 for g in range(NG)]
        keeprel = [bits[g][1] * 512 for g in range(NG)]
        f2 = [(1 - bits[g][2]) * 256 for g in range(NG)]
        k2 = [bits[g][2] * 256 for g in range(NG)]
        starts1 = [starts0[g] + keeprel[g] for g in range(NG)]
        starts2 = [starts1[g] + k2[g] for g in range(NG)]
        fwd2rel = [keeprel[g] + f2[g] for g in range(NG)]
        keep2rel = [keeprel[g] + k2[g] for g in range(NG)]

        barrier_sem = pltpu.get_barrier_semaphore()
        for s in range(3):
            pl.semaphore_signal(
                barrier_sem,
                inc=1,
                device_id=(_gray(r ^ (1 << s)),),
                device_id_type=pl.DeviceIdType.MESH,
            )
        pl.semaphore_wait(barrier_sem, 3)

        b_bf16 = b_ref[:, :].astype(jnp.bfloat16)

        def _mm(row_start, nrows, g):
            c0, c1 = cols[g]
            return jnp.dot(
                a_ref[pl.ds(row_start, nrows), :].astype(jnp.bfloat16),
                b_bf16[:, c0:c1],
                preferred_element_type=jnp.float32,
            )

        def _rs(g, slot, stage_rows, src_rows, dst_row, nrows):
            rd = pltpu.make_async_remote_copy(
                src_ref=sends[g].at[src_rows[0] : src_rows[1], :],
                dst_ref=recvs[g].at[pl.ds(dst_row, nrows), :],
                send_sem=rs_send.at[slot, g],
                recv_sem=rs_recv.at[slot, g],
                device_id=(nbrs[g][stage_rows],),
                device_id_type=pl.DeviceIdType.MESH,
            )
            rd.start()
            return rd

        p0 = [None] * NG
        for g in range(NG):
            c0, c1 = cols[g]
            rows = send_base[g] + fwdrel[g]
            sends[g][0:512, :] = _mm(rows, 512, g).astype(jnp.bfloat16)
            p0[g] = _rs(g, 0, 0, (0, 512), fwdrel[g], 512)

        for g in range(NG):
            c0, c1 = cols[g]
            acc[pl.ds(starts0[g] + fwdrel[g], 512), c0:c1] = _mm(
                starts0[g] + fwdrel[g], 512, g
            )

        p3 = [None] * NG
        p4 = [None] * NG
        for g in range(NG):
            c0, c1 = cols[g]
            p0[g].wait()
            base = starts0[g] + fwdrel[g]
            pa = acc[pl.ds(base + f2[g], 256), c0:c1] + recvs[g][
                pl.ds(fwdrel[g] + f2[g], 256), :
            ].astype(jnp.float32)
            sends[g][0:256, :] = pa.astype(jnp.bfloat16)
            p3[g] = _rs(g, 3, 1, (0, 256), T1_OFF + f2[g], 256)
            pb = acc[pl.ds(base + k2[g], 256), c0:c1] + recvs[g][
                pl.ds(fwdrel[g] + k2[g], 256), :
            ].astype(jnp.float32)
            sends[g][256:512, :] = pb.astype(jnp.bfloat16)
            p4[g] = _rs(g, 4, 1, (256, 512), T1_OFF + k2[g], 256)

        p1 = [None] * NG
        p2 = [None] * NG
        for g in range(NG):
            sends[g][512:768, :] = _mm(
                send_base[g] + fwd2rel[g], 256, g
            ).astype(jnp.bfloat16)
            p1[g] = _rs(g, 1, 0, (512, 768), fwd2rel[g], 256)
            sends[g][768:1024, :] = _mm(
                send_base[g] + keep2rel[g], 256, g
            ).astype(jnp.bfloat16)
            p2[g] = _rs(g, 2, 0, (768, 1024), keep2rel[g], 256)

        for g in range(NG):
            c0, c1 = cols[g]
            acc[pl.ds(starts1[g], 512), c0:c1] = _mm(starts1[g], 512, g)

        for g in range(NG):
            c0, c1 = cols[g]
            p1[g].wait()
            rows = starts1[g] + f2[g]
            acc[pl.ds(rows, 256), c0:c1] = acc[pl.ds(rows, 256), c0:c1] + recvs[
                g
            ][pl.ds(fwd2rel[g], 256), :].astype(jnp.float32)

        p5 = [None] * NG
        for g in range(NG):
            c0, c1 = cols[g]
            p3[g].wait()
            rows = starts1[g] + f2[g]
            s2 = acc[pl.ds(rows, 256), c0:c1] + recvs[g][
                pl.ds(T1_OFF + f2[g], 256), :
            ].astype(jnp.float32)
            sends[g][0:256, :] = s2.astype(jnp.bfloat16)
            p5[g] = _rs(g, 5, 2, (0, 256), T2_OFF, 256)

        for g in range(NG):
            c0, c1 = cols[g]
            p2[g].wait()
            acc[pl.ds(starts2[g], 256), c0:c1] = acc[
                pl.ds(starts2[g], 256), c0:c1
            ] + recvs[g][pl.ds(keep2rel[g], 256), :].astype(jnp.float32)
        for g in range(NG):
            c0, c1 = cols[g]
            p4[g].wait()
            acc[pl.ds(starts2[g], 256), c0:c1] = acc[
                pl.ds(starts2[g], 256), c0:c1
            ] + recvs[g][pl.ds(T1_OFF + k2[g], 256), :].astype(jnp.float32)

        ag_rd = [[None] * 7 for _ in range(NG)]

        def _ag_piece(g, idx, row_start, nrows, target):
            c0, c1 = cols[g]
            rd = pltpu.make_async_remote_copy(
                src_ref=out_ref.at[pl.ds(row_start, nrows), c0:c1],
                dst_ref=out_ref.at[pl.ds(row_start, nrows), c0:c1],
                send_sem=ag_send.at[idx, g],
                recv_sem=ag_recv.at[idx, g],
                device_id=(target,),
                device_id_type=pl.DeviceIdType.MESH,
            )
            rd.start()
            ag_rd[g][idx] = rd

        for g in range(NG):
            c0, c1 = cols[g]
            p5[g].wait()
            z = acc[pl.ds(starts2[g], CHUNK), c0:c1] + recvs[g][
                pl.ds(T2_OFF, CHUNK), :
            ].astype(jnp.float32)
            gz = 0.5 * z * (1.0 + jnp.tanh(0.7978845608 * (z + 0.044715 * z * z * z)))
            out_ref[pl.ds(starts2[g], CHUNK), c0:c1] = gz.astype(jnp.bfloat16)
            _ag_piece(g, 0, starts2[g], CHUNK, nbrs[g][2])
            _ag_piece(g, 1, starts2[g], CHUNK, nbrs[g][1])
            _ag_piece(g, 2, starts2[g], CHUNK, nbrs[g][0])

        for g in range(NG):
            sib_a2 = starts2[g] + (1 - 2 * bits[g][2]) * 256
            ag_rd[g][0].wait()
            _ag_piece(g, 3, sib_a2, CHUNK, nbrs[g][1])
            _ag_piece(g, 4, sib_a2, CHUNK, nbrs[g][0])

        for g in range(NG):
            sib_a1 = starts2[g] + (1 - 2 * bits[g][1]) * 512
            ag_rd[g][1].wait()
            _ag_piece(g, 5, sib_a1, CHUNK, nbrs[g][0])

        for g in range(NG):
            sib_a1a2 = (
                starts2[g]
                + (1 - 2 * bits[g][1]) * 512
                + (1 - 2 * bits[g][2]) * 256
            )
            ag_rd[g][3].wait()
            _ag_piece(g, 6, sib_a1a2, CHUNK, nbrs[g][0])

        for g in range(NG):
            ag_rd[g][2].wait()
            ag_rd[g][4].wait()
            ag_rd[g][5].wait()
            ag_rd[g][6].wait()

    wg = [COL_BOUNDS[g + 1] - COL_BOUNDS[g] for g in range(NG)]
    return pl.pallas_call(
        body,
        out_shape=jax.ShapeDtypeStruct((M, N), jnp.bfloat16),
        in_specs=[
            pl.BlockSpec(memory_space=pltpu.VMEM),
            pl.BlockSpec(memory_space=pltpu.VMEM),
        ],
        out_specs=pl.BlockSpec(memory_space=pltpu.VMEM),
        scratch_shapes=[
            pltpu.VMEM((M, N), jnp.float32),
            pltpu.VMEM((M // 2, wg[0]), jnp.bfloat16),
            pltpu.VMEM((M // 2, wg[1]), jnp.bfloat16),
            pltpu.VMEM((M // 2, wg[2]), jnp.bfloat16),
            pltpu.VMEM((1792, wg[0]), jnp.bfloat16),
            pltpu.VMEM((1792, wg[1]), jnp.bfloat16),
            pltpu.VMEM((1792, wg[2]), jnp.bfloat16),
            pltpu.SemaphoreType.DMA((6, 3)),
            pltpu.SemaphoreType.DMA((6, 3)),
            pltpu.SemaphoreType.DMA((7, 3)),
            pltpu.SemaphoreType.DMA((7, 3)),
        ],
        compiler_params=pltpu.CompilerParams(
            vmem_limit_bytes=100 * 1024 * 1024,
            collective_id=0,
        ),
    )(A, B)


# device time: 81000 ns/iter; 4.5302x vs baseline; 1.0122x over previous
import jax
import jax.numpy as jnp
from jax import lax
from jax.experimental import pallas as pl
from jax.experimental.pallas import tpu as pltpu

N_DEV = 8
M = 2048
N = 2048
CHUNK = M // N_DEV

COL_BOUNDS = (0, 768, 1408, 2048)
NG = 3
T1_OFF = 1024
T2_OFF = 1536


def _gray(p):
    return p ^ ((p >> 1) & 1)


def kernel(A, B):
    m, k_per = A.shape
    _, n = B.shape
    assert (m, n) == (M, N)

    def body(
        a_ref,
        b_ref,
        out_ref,
        acc,
        send0,
        send1,
        send2,
        recv0,
        recv1,
        recv2,
        rs_send,
        rs_recv,
        ag_send,
        ag_recv,
    ):
        my_pos = lax.axis_index("i")
        r = _gray(my_pos)
        sends = [send0, send1, send2]
        recvs = [recv0, recv1, recv2]
        cols = [(COL_BOUNDS[g], COL_BOUNDS[g + 1]) for g in range(NG)]
        bits = [[(r >> ((g + t) % 3)) & 1 for t in range(3)] for g in range(NG)]
        nbrs = [
            [_gray(r ^ (1 << ((g + t) % 3))) for t in range(3)] for g in range(NG)
        ]
        send_base = [(1 - bits[g][0]) * (M // 2) for g in range(NG)]
        starts0 = [bits[g][0] * (M // 2) for g in range(NG)]
        fwdrel = [(1 - bits[g][1]) * 512 for g in range(NG)]
        keeprel = [bits[g][1] * 512 for g in range(NG)]
        f2 = [(1 - bits[g][2]) * 256 for g in range(NG)]
        k2 = [bits[g][2] * 256 for g in range(NG)]
        starts1 = [starts0[g] + keeprel[g] for g in range(NG)]
        starts2 = [starts1[g] + k2[g] for g in range(NG)]
        fwd2rel = [keeprel[g] + f2[g] for g in range(NG)]
        keep2rel = [keeprel[g] + k2[g] for g in range(NG)]

        barrier_sem = pltpu.get_barrier_semaphore()
        for s in range(3):
            pl.semaphore_signal(
                barrier_sem,
                inc=1,
                device_id=(_gray(r ^ (1 << s)),),
                device_id_type=pl.DeviceIdType.MESH,
            )
        pl.semaphore_wait(barrier_sem, 3)

        b_bf16 = b_ref[:, :].astype(jnp.bfloat16)

        def _mm(row_start, nrows, g):
            c0, c1 = cols[g]
            return jnp.dot(
                a_ref[pl.ds(row_start, nrows), :].astype(jnp.bfloat16),
                b_bf16[:, c0:c1],
                preferred_element_type=jnp.float32,
            )

        def _rs(g, slot, stage_rows, src_rows, dst_row, nrows):
            rd = pltpu.make_async_remote_copy(
                src_ref=sends[g].at[src_rows[0] : src_rows[1], :],
                dst_ref=recvs[g].at[pl.ds(dst_row, nrows), :],
                send_sem=rs_send.at[slot, g],
                recv_sem=rs_recv.at[slot, g],
                device_id=(nbrs[g][stage_rows],),
                device_id_type=pl.DeviceIdType.MESH,
            )
            rd.start()
            return rd

        p0 = [None] * NG
        for g in range(NG):
            c0, c1 = cols[g]
            rows = send_base[g] + fwdrel[g]
            sends[g][0:512, :] = _mm(rows, 512, g).astype(jnp.bfloat16)
            p0[g] = _rs(g, 0, 0, (0, 512), fwdrel[g], 512)

        for g in range(NG):
            c0, c1 = cols[g]
            acc[pl.ds(starts0[g] + fwdrel[g], 512), c0:c1] = _mm(
                starts0[g] + fwdrel[g], 512, g
            )

        p3 = [None] * NG
        p4 = [None] * NG
        for g in range(NG):
            c0, c1 = cols[g]
            p0[g].wait()
            base = starts0[g] + fwdrel[g]
            pa = acc[pl.ds(base + f2[g], 256), c0:c1] + recvs[g][
                pl.ds(fwdrel[g] + f2[g], 256), :
            ].astype(jnp.float32)
            sends[g][0:256, :] = pa.astype(jnp.bfloat16)
            p3[g] = _rs(g, 3, 1, (0, 256), T1_OFF + f2[g], 256)
            pb = acc[pl.ds(base + k2[g], 256), c0:c1] + recvs[g][
                pl.ds(fwdrel[g] + k2[g], 256), :
            ].astype(jnp.float32)
            sends[g][256:512, :] = pb.astype(jnp.bfloat16)
            p4[g] = _rs(g, 4, 1, (256, 512), T1_OFF + k2[g], 256)

        p1 = [None] * NG
        p2 = [None] * NG
        for g in range(NG):
            sends[g][512:768, :] = _mm(
                send_base[g] + fwd2rel[g], 256, g
            ).astype(jnp.bfloat16)
            p1[g] = _rs(g, 1, 0, (512, 768), fwd2rel[g], 256)
            sends[g][768:1024, :] = _mm(
                send_base[g] + keep2rel[g], 256, g
            ).astype(jnp.bfloat16)
            p2[g] = _rs(g, 2, 0, (768, 1024), keep2rel[g], 256)

        for g in range(NG):
            c0, c1 = cols[g]
            acc[pl.ds(starts1[g], 512), c0:c1] = _mm(starts1[g], 512, g)

        for g in range(NG):
            c0, c1 = cols[g]
            p1[g].wait()
            rows = starts1[g] + f2[g]
            acc[pl.ds(rows, 256), c0:c1] = acc[pl.ds(rows, 256), c0:c1] + recvs[
                g
            ][pl.ds(fwd2rel[g], 256), :].astype(jnp.float32)

        p5 = [None] * NG
        for g in range(NG):
            c0, c1 = cols[g]
            p3[g].wait()
            rows = starts1[g] + f2[g]
            s2 = acc[pl.ds(rows, 256), c0:c1] + recvs[g][
                pl.ds(T1_OFF + f2[g], 256), :
            ].astype(jnp.float32)
            sends[g][0:256, :] = s2.astype(jnp.bfloat16)
            p5[g] = _rs(g, 5, 2, (0, 256), T2_OFF, 256)

        for g in range(NG):
            c0, c1 = cols[g]
            p2[g].wait()
            p4[g].wait()
            acc[pl.ds(starts2[g], 256), c0:c1] = (
                acc[pl.ds(starts2[g], 256), c0:c1]
                + recvs[g][pl.ds(keep2rel[g], 256), :].astype(jnp.float32)
                + recvs[g][pl.ds(T1_OFF + k2[g], 256), :].astype(jnp.float32)
            )

        ag_rd = [[None] * 7 for _ in range(NG)]

        def _ag_piece(g, idx, row_start, nrows, target):
            c0, c1 = cols[g]
            rd = pltpu.make_async_remote_copy(
                src_ref=out_ref.at[pl.ds(row_start, nrows), c0:c1],
                dst_ref=out_ref.at[pl.ds(row_start, nrows), c0:c1],
                send_sem=ag_send.at[idx, g],
                recv_sem=ag_recv.at[idx, g],
                device_id=(target,),
                device_id_type=pl.DeviceIdType.MESH,
            )
            rd.start()
            ag_rd[g][idx] = rd

        for g in range(NG):
            c0, c1 = cols[g]
            p5[g].wait()
            z = acc[pl.ds(starts2[g], CHUNK), c0:c1] + recvs[g][
                pl.ds(T2_OFF, CHUNK), :
            ].astype(jnp.float32)
            gz = 0.5 * z * (1.0 + jnp.tanh(0.7978845608 * (z + 0.044715 * z * z * z)))
            out_ref[pl.ds(starts2[g], CHUNK), c0:c1] = gz.astype(jnp.bfloat16)
            _ag_piece(g, 2, starts2[g], CHUNK, nbrs[g][0])
            _ag_piece(g, 0, starts2[g], CHUNK, nbrs[g][2])
            _ag_piece(g, 1, starts2[g], CHUNK, nbrs[g][1])

        for g in range(NG):
            sib_a2 = starts2[g] + (1 - 2 * bits[g][2]) * 256
            ag_rd[g][0].wait()
            _ag_piece(g, 3, sib_a2, CHUNK, nbrs[g][1])
            _ag_piece(g, 4, sib_a2, CHUNK, nbrs[g][0])

        for g in range(NG):
            sib_a1 = starts2[g] + (1 - 2 * bits[g][1]) * 512
            ag_rd[g][1].wait()
            _ag_piece(g, 5, sib_a1, CHUNK, nbrs[g][0])

        for g in range(NG):
            sib_a1a2 = (
                starts2[g]
                + (1 - 2 * bits[g][1]) * 512
                + (1 - 2 * bits[g][2]) * 256
            )
            ag_rd[g][3].wait()
            _ag_piece(g, 6, sib_a1a2, CHUNK, nbrs[g][0])

        for g in range(NG):
            ag_rd[g][2].wait()
            ag_rd[g][4].wait()
            ag_rd[g][5].wait()
            ag_rd[g][6].wait()

    wg = [COL_BOUNDS[g + 1] - COL_BOUNDS[g] for g in range(NG)]
    return pl.pallas_call(
        body,
        out_shape=jax.ShapeDtypeStruct((M, N), jnp.bfloat16),
        in_specs=[
            pl.BlockSpec(memory_space=pltpu.VMEM),
            pl.BlockSpec(memory_space=pltpu.VMEM),
        ],
        out_specs=pl.BlockSpec(memory_space=pltpu.VMEM),
        scratch_shapes=[
            pltpu.VMEM((M, N), jnp.float32),
            pltpu.VMEM((M // 2, wg[0]), jnp.bfloat16),
            pltpu.VMEM((M // 2, wg[1]), jnp.bfloat16),
            pltpu.VMEM((M // 2, wg[2]), jnp.bfloat16),
            pltpu.VMEM((1792, wg[0]), jnp.bfloat16),
            pltpu.VMEM((1792, wg[1]), jnp.bfloat16),
            pltpu.VMEM((1792, wg[2]), jnp.bfloat16),
            pltpu.SemaphoreType.DMA((6, 3)),
            pltpu.SemaphoreType.DMA((6, 3)),
            pltpu.SemaphoreType.DMA((7, 3)),
            pltpu.SemaphoreType.DMA((7, 3)),
        ],
        compiler_params=pltpu.CompilerParams(
            vmem_limit_bytes=100 * 1024 * 1024,
            collective_id=0,
        ),
    )(A, B)


# device time: 80016 ns/iter; 4.5860x vs baseline; 1.0123x over previous
import jax
import jax.numpy as jnp
from jax import lax
from jax.experimental import pallas as pl
from jax.experimental.pallas import tpu as pltpu

N_DEV = 8
M = 2048
N = 2048
CHUNK = M // N_DEV

COL_BOUNDS = (0, 768, 1408, 2048)
NG = 3
T1_OFF = 1024
T2_OFF = 1536


def _gray(p):
    return p ^ ((p >> 1) & 1)


def kernel(A, B):
    m, k_per = A.shape
    _, n = B.shape
    assert (m, n) == (M, N)

    def body(
        a_ref,
        b_ref,
        out_ref,
        acc,
        send0,
        send1,
        send2,
        recv0,
        recv1,
        recv2,
        rs_send,
        rs_recv,
        ag_send,
        ag_recv,
    ):
        my_pos = lax.axis_index("i")
        r = _gray(my_pos)
        sends = [send0, send1, send2]
        recvs = [recv0, recv1, recv2]
        cols = [(COL_BOUNDS[g], COL_BOUNDS[g + 1]) for g in range(NG)]
        bits = [[(r >> ((g + t) % 3)) & 1 for t in range(3)] for g in range(NG)]
        nbrs = [
            [_gray(r ^ (1 << ((g + t) % 3))) for t in range(3)] for g in range(NG)
        ]
        send_base = [(1 - bits[g][0]) * (M // 2) for g in range(NG)]
        starts0 = [bits[g][0] * (M // 2) for g in range(NG)]
        fwdrel = [(1 - bits[g][1]) * 512 for g in range(NG)]
        keeprel = [bits[g][1] * 512 for g in range(NG)]
        f2 = [(1 - bits[g][2]) * 256 for g in range(NG)]
        k2 = [bits[g][2] * 256 for g in range(NG)]
        starts1 = [starts0[g] + keeprel[g] for g in range(NG)]
        starts2 = [starts1[g] + k2[g] for g in range(NG)]
        fwd2rel = [keeprel[g] + f2[g] for g in range(NG)]
        keep2rel = [keeprel[g] + k2[g] for g in range(NG)]

        barrier_sem = pltpu.get_barrier_semaphore()
        for s in range(3):
            pl.semaphore_signal(
                barrier_sem,
                inc=1,
                device_id=(_gray(r ^ (1 << s)),),
                device_id_type=pl.DeviceIdType.MESH,
            )
        pl.semaphore_wait(barrier_sem, 3)

        b_bf16 = b_ref[:, :].astype(jnp.bfloat16)

        def _mm(row_start, nrows, g):
            c0, c1 = cols[g]
            return jnp.dot(
                a_ref[pl.ds(row_start, nrows), :].astype(jnp.bfloat16),
                b_bf16[:, c0:c1],
                preferred_element_type=jnp.float32,
            )

        def _rs(g, slot, stage_rows, src_rows, dst_row, nrows):
            rd = pltpu.make_async_remote_copy(
                src_ref=sends[g].at[src_rows[0] : src_rows[1], :],
                dst_ref=recvs[g].at[pl.ds(dst_row, nrows), :],
                send_sem=rs_send.at[slot, g],
                recv_sem=rs_recv.at[slot, g],
                device_id=(nbrs[g][stage_rows],),
                device_id_type=pl.DeviceIdType.MESH,
            )
            rd.start()
            return rd

        p0a = [None] * NG
        p0b = [None] * NG
        for g in range(NG):
            rows = send_base[g] + fwdrel[g]
            sends[g][0:256, :] = _mm(rows + f2[g], 256, g).astype(jnp.bfloat16)
            p0a[g] = _rs(g, 0, 0, (0, 256), fwdrel[g] + f2[g], 256)
        for g in range(NG):
            rows = send_base[g] + fwdrel[g]
            sends[g][256:512, :] = _mm(rows + k2[g], 256, g).astype(jnp.bfloat16)
            p0b[g] = _rs(g, 6, 0, (256, 512), fwdrel[g] + k2[g], 256)

        for g in range(NG):
            c0, c1 = cols[g]
            acc[pl.ds(starts0[g] + fwdrel[g], 512), c0:c1] = _mm(
                starts0[g] + fwdrel[g], 512, g
            )

        p3 = [None] * NG
        p4 = [None] * NG
        for g in range(NG):
            c0, c1 = cols[g]
            p0a[g].wait()
            base = starts0[g] + fwdrel[g]
            pa = acc[pl.ds(base + f2[g], 256), c0:c1] + recvs[g][
                pl.ds(fwdrel[g] + f2[g], 256), :
            ].astype(jnp.float32)
            sends[g][0:256, :] = pa.astype(jnp.bfloat16)
            p3[g] = _rs(g, 3, 1, (0, 256), T1_OFF + f2[g], 256)
        for g in range(NG):
            c0, c1 = cols[g]
            p0b[g].wait()
            base = starts0[g] + fwdrel[g]
            pb = acc[pl.ds(base + k2[g], 256), c0:c1] + recvs[g][
                pl.ds(fwdrel[g] + k2[g], 256), :
            ].astype(jnp.float32)
            sends[g][256:512, :] = pb.astype(jnp.bfloat16)
            p4[g] = _rs(g, 4, 1, (256, 512), T1_OFF + k2[g], 256)

        p1 = [None] * NG
        p2 = [None] * NG
        for g in range(NG):
            sends[g][512:768, :] = _mm(
                send_base[g] + fwd2rel[g], 256, g
            ).astype(jnp.bfloat16)
            p1[g] = _rs(g, 1, 0, (512, 768), fwd2rel[g], 256)
            sends[g][768:1024, :] = _mm(
                send_base[g] + keep2rel[g], 256, g
            ).astype(jnp.bfloat16)
            p2[g] = _rs(g, 2, 0, (768, 1024), keep2rel[g], 256)

        for g in range(NG):
            c0, c1 = cols[g]
            acc[pl.ds(starts1[g], 512), c0:c1] = _mm(starts1[g], 512, g)

        for g in range(NG):
            c0, c1 = cols[g]
            p1[g].wait()
            rows = starts1[g] + f2[g]
            acc[pl.ds(rows, 256), c0:c1] = acc[pl.ds(rows, 256), c0:c1] + recvs[
                g
            ][pl.ds(fwd2rel[g], 256), :].astype(jnp.float32)

        p5 = [None] * NG
        for g in range(NG):
            c0, c1 = cols[g]
            p3[g].wait()
            rows = starts1[g] + f2[g]
            s2 = acc[pl.ds(rows, 256), c0:c1] + recvs[g][
                pl.ds(T1_OFF + f2[g], 256), :
            ].astype(jnp.float32)
            sends[g][0:256, :] = s2.astype(jnp.bfloat16)
            p5[g] = _rs(g, 5, 2, (0, 256), T2_OFF, 256)

        for g in range(NG):
            c0, c1 = cols[g]
            p2[g].wait()
            p4[g].wait()
            acc[pl.ds(starts2[g], 256), c0:c1] = (
                acc[pl.ds(starts2[g], 256), c0:c1]
                + recvs[g][pl.ds(keep2rel[g], 256), :].astype(jnp.float32)
                + recvs[g][pl.ds(T1_OFF + k2[g], 256), :].astype(jnp.float32)
            )

        ag_rd = [[None] * 7 for _ in range(NG)]

        def _ag_piece(g, idx, row_start, nrows, target):
            c0, c1 = cols[g]
            rd = pltpu.make_async_remote_copy(
                src_ref=out_ref.at[pl.ds(row_start, nrows), c0:c1],
                dst_ref=out_ref.at[pl.ds(row_start, nrows), c0:c1],
                send_sem=ag_send.at[idx, g],
                recv_sem=ag_recv.at[idx, g],
                device_id=(target,),
                device_id_type=pl.DeviceIdType.MESH,
            )
            rd.start()
            ag_rd[g][idx] = rd

        for g in range(NG):
            c0, c1 = cols[g]
            p5[g].wait()
            z = acc[pl.ds(starts2[g], CHUNK), c0:c1] + recvs[g][
                pl.ds(T2_OFF, CHUNK), :
            ].astype(jnp.float32)
            gz = 0.5 * z * (1.0 + jnp.tanh(0.7978845608 * (z + 0.044715 * z * z * z)))
            out_ref[pl.ds(starts2[g], CHUNK), c0:c1] = gz.astype(jnp.bfloat16)
            _ag_piece(g, 2, starts2[g], CHUNK, nbrs[g][0])
            _ag_piece(g, 0, starts2[g], CHUNK, nbrs[g][2])
            _ag_piece(g, 1, starts2[g], CHUNK, nbrs[g][1])

        for g in range(NG):
            sib_a2 = starts2[g] + (1 - 2 * bits[g][2]) * 256
            ag_rd[g][0].wait()
            _ag_piece(g, 3, sib_a2, CHUNK, nbrs[g][1])
            _ag_piece(g, 4, sib_a2, CHUNK, nbrs[g][0])

        for g in range(NG):
            sib_a1 = starts2[g] + (1 - 2 * bits[g][1]) * 512
            ag_rd[g][1].wait()
            _ag_piece(g, 5, sib_a1, CHUNK, nbrs[g][0])

        for g in range(NG):
            sib_a1a2 = (
                starts2[g]
                + (1 - 2 * bits[g][1]) * 512
                + (1 - 2 * bits[g][2]) * 256
            )
            ag_rd[g][3].wait()
            _ag_piece(g, 6, sib_a1a2, CHUNK, nbrs[g][0])

        for g in range(NG):
            ag_rd[g][2].wait()
            ag_rd[g][4].wait()
            ag_rd[g][5].wait()
            ag_rd[g][6].wait()

    wg = [COL_BOUNDS[g + 1] - COL_BOUNDS[g] for g in range(NG)]
    return pl.pallas_call(
        body,
        out_shape=jax.ShapeDtypeStruct((M, N), jnp.bfloat16),
        in_specs=[
            pl.BlockSpec(memory_space=pltpu.VMEM),
            pl.BlockSpec(memory_space=pltpu.VMEM),
        ],
        out_specs=pl.BlockSpec(memory_space=pltpu.VMEM),
        scratch_shapes=[
            pltpu.VMEM((M, N), jnp.float32),
            pltpu.VMEM((M // 2, wg[0]), jnp.bfloat16),
            pltpu.VMEM((M // 2, wg[1]), jnp.bfloat16),
            pltpu.VMEM((M // 2, wg[2]), jnp.bfloat16),
            pltpu.VMEM((1792, wg[0]), jnp.bfloat16),
            pltpu.VMEM((1792, wg[1]), jnp.bfloat16),
            pltpu.VMEM((1792, wg[2]), jnp.bfloat16),
            pltpu.SemaphoreType.DMA((7, 3)),
            pltpu.SemaphoreType.DMA((7, 3)),
            pltpu.SemaphoreType.DMA((7, 3)),
            pltpu.SemaphoreType.DMA((7, 3)),
        ],
        compiler_params=pltpu.CompilerParams(
            vmem_limit_bytes=100 * 1024 * 1024,
            collective_id=0,
        ),
    )(A, B)
